# Initial kernel scaffold; baseline (speedup 1.0000x reference)
import jax
import jax.numpy as jnp
from jax.experimental import pallas as pl


def kernel(x, anchors, codebook):
    raise NotImplementedError("write your pallas kernel here")


# speedup vs baseline: 1.8294x; 1.8294x over previous
import functools

import jax
import jax.numpy as jnp
from jax import lax
from jax.experimental import pallas as pl
from jax.experimental.pallas import tpu as pltpu
from jax.experimental.pallas import tpu_sc as plsc

N = 8192
K = 8192
D = 256

BN = 256
BK = 512
NI = N // BN
NJ = K // BK

def _argmax_body(x_ref, at_ref, out_ref, vmax_ref, vidx_ref):
    j = pl.program_id(1)

    @pl.when(j == 0)
    def _init():
        vmax_ref[...] = jnp.full((BN, BK), -jnp.inf, jnp.float32)
        vidx_ref[...] = jnp.zeros((BN, BK), jnp.int32)

    x0 = x_ref[:, 0:1]
    x1 = x_ref[:, 1:2]
    x2 = x_ref[:, 2:3]
    a0 = at_ref[0:1, :]
    a1 = at_ref[1:2, :]
    a2 = at_ref[2:3, :]
    d0 = x0 - a0
    d1 = x1 - a1
    d2 = x2 - a2
    dist2 = d0 * d0 + d1 * d1 + d2 * d2

    cur = vmax_ref[...]
    upd = dist2 > cur
    col = lax.broadcasted_iota(jnp.int32, (BN, BK), 1) + j * BK
    vmax_ref[...] = jnp.where(upd, dist2, cur)
    vidx_ref[...] = jnp.where(upd, col, vidx_ref[...])

    @pl.when(j == NJ - 1)
    def _finish():
        vm = vmax_ref[...]
        vi = vidx_ref[...]
        m = jnp.max(vm, axis=1, keepdims=True)
        idx = jnp.min(jnp.where(vm == m, vi, 2**31 - 1), axis=1)
        out_ref[...] = idx.reshape(1, 1, BN)


_argmax_call = pl.pallas_call(
    _argmax_body,
    grid=(NI, NJ),
    in_specs=[
        pl.BlockSpec((BN, 3), lambda i, j: (i, 0)),
        pl.BlockSpec((3, BK), lambda i, j: (0, j)),
    ],
    out_specs=pl.BlockSpec((1, 1, BN), lambda i, j: (i, 0, 0)),
    out_shape=jax.ShapeDtypeStruct((NI, 1, BN), jnp.int32),
    scratch_shapes=[
        pltpu.VMEM((BN, BK), jnp.float32),
        pltpu.VMEM((BN, BK), jnp.int32),
    ],
    compiler_params=pltpu.CompilerParams(
        dimension_semantics=("parallel", "arbitrary"),
    ),
)


_NC = 2
_NS = 16
_NW = _NC * _NS
BPW = N // _NW

_sc_mesh = plsc.VectorSubcoreMesh(core_axis_name="c", subcore_axis_name="s")


@functools.partial(
    pl.kernel,
    mesh=_sc_mesh,
    out_type=jax.ShapeDtypeStruct((N, D), jnp.float32),
    scratch_types=[
        pltpu.VMEM((BPW,), jnp.int32),
        pltpu.VMEM((BPW, D), jnp.float32),
        pltpu.SemaphoreType.DMA,
    ],
)
def _sc_gather(table_hbm, idx_hbm, out_hbm, idx_v, rows_v, sem):
    wid = lax.axis_index("s") * _NC + lax.axis_index("c")
    base = wid * BPW
    pltpu.sync_copy(idx_hbm.at[pl.ds(base, BPW)], idx_v)
    pltpu.async_copy(table_hbm.at[idx_v], rows_v, sem).wait()
    pltpu.sync_copy(rows_v, out_hbm.at[pl.ds(base, BPW)])


def kernel(x, anchors, codebook):
    at = anchors.T
    idx = _argmax_call(x, at).reshape(N)
    return _sc_gather(codebook, idx)


# speedup vs baseline: 2.6639x; 1.4562x over previous
import functools

import jax
import jax.numpy as jnp
from jax import lax
from jax.experimental import pallas as pl
from jax.experimental.pallas import tpu as pltpu
from jax.experimental.pallas import tpu_sc as plsc

N = 8192
K = 8192
D = 256

BN = 512
BK = 512
NI = N // BN
NJ = K // BK

def _argmax_body(x_ref, at_ref, out_ref, vmax_ref, vidx_ref):
    j = pl.program_id(1)

    @pl.when(j == 0)
    def _init():
        vmax_ref[...] = jnp.full((BN, BK), -jnp.inf, jnp.float32)
        vidx_ref[...] = jnp.zeros((BN, BK), jnp.int32)

    x0 = x_ref[:, 0:1]
    x1 = x_ref[:, 1:2]
    x2 = x_ref[:, 2:3]
    a0 = at_ref[0:1, :]
    a1 = at_ref[1:2, :]
    a2 = at_ref[2:3, :]
    d0 = x0 - a0
    d1 = x1 - a1
    d2 = x2 - a2
    dist2 = d0 * d0 + d1 * d1 + d2 * d2

    cur = vmax_ref[...]
    upd = dist2 > cur
    vmax_ref[...] = jnp.maximum(dist2, cur)
    vidx_ref[...] = jnp.where(upd, j, vidx_ref[...])

    @pl.when(j == NJ - 1)
    def _finish():
        vm = vmax_ref[...]
        lane = lax.broadcasted_iota(jnp.int32, (BN, BK), 1)
        vi = vidx_ref[...] * BK + lane
        m = jnp.max(vm, axis=1, keepdims=True)
        idx = jnp.min(jnp.where(vm == m, vi, 2**31 - 1), axis=1)
        out_ref[...] = idx.reshape(1, 1, BN)


_argmax_call = pl.pallas_call(
    _argmax_body,
    grid=(NI, NJ),
    in_specs=[
        pl.BlockSpec((BN, 3), lambda i, j: (i, 0)),
        pl.BlockSpec((3, BK), lambda i, j: (0, j)),
    ],
    out_specs=pl.BlockSpec((1, 1, BN), lambda i, j: (i, 0, 0)),
    out_shape=jax.ShapeDtypeStruct((NI, 1, BN), jnp.int32),
    scratch_shapes=[
        pltpu.VMEM((BN, BK), jnp.float32),
        pltpu.VMEM((BN, BK), jnp.int32),
    ],
    compiler_params=pltpu.CompilerParams(
        dimension_semantics=("parallel", "arbitrary"),
    ),
)


_NC = 2
_NS = 16
_NW = _NC * _NS
BPW = N // _NW

_sc_mesh = plsc.VectorSubcoreMesh(core_axis_name="c", subcore_axis_name="s")


@functools.partial(
    pl.kernel,
    mesh=_sc_mesh,
    out_type=jax.ShapeDtypeStruct((N, D), jnp.float32),
    scratch_types=[
        pltpu.VMEM((BPW,), jnp.int32),
        pltpu.VMEM((BPW, D), jnp.float32),
        pltpu.SemaphoreType.DMA,
    ],
)
def _sc_gather(table_hbm, idx_hbm, out_hbm, idx_v, rows_v, sem):
    wid = lax.axis_index("s") * _NC + lax.axis_index("c")
    base = wid * BPW
    pltpu.sync_copy(idx_hbm.at[pl.ds(base, BPW)], idx_v)
    pltpu.async_copy(table_hbm.at[idx_v], rows_v, sem).wait()
    pltpu.sync_copy(rows_v, out_hbm.at[pl.ds(base, BPW)])


def kernel(x, anchors, codebook):
    at = anchors.T
    idx = _argmax_call(x, at).reshape(N)
    return _sc_gather(codebook, idx)
